# baseline (speedup 1.0000x reference)
import functools

import jax
import jax.numpy as jnp
from jax.experimental import pallas as pl
from jax.experimental.pallas import tpu as pltpu

TOPK_N = 8
Q_BLOCK = 512
K_BLOCK = 2048
LANES = 128
GROUPS = K_BLOCK // LANES
NSLOT = 26
NEG_INF = float("-inf")
INT_MAX = 2**31 - 1


def _insert(rv, rt, base, depth, v, t):
    c = [v > rv[base + s] for s in range(depth)]
    for s in range(depth - 1, 0, -1):
        rv[base + s] = jnp.where(c[s], jnp.where(c[s - 1], rv[base + s - 1], v),
                                 rv[base + s])
        rt[base + s] = jnp.where(c[s], jnp.where(c[s - 1], rt[base + s - 1], t),
                                 rt[base + s])
    rv[base] = jnp.where(c[0], v, rv[base])
    rt[base] = jnp.where(c[0], t, rt[base])


def _fused_body(k_total, q_ref, e_ref, vals_ref, idx_ref, qn_ref, rv_ref, rt_ref):
    k = pl.program_id(1)
    nk = pl.num_programs(1)

    @pl.when(k == 0)
    def _init():
        q = q_ref[...]
        denom = jnp.clip(jnp.sum(jnp.abs(q), axis=1, keepdims=True), 1e-12, None)
        qn_ref[...] = q / denom
        rv_ref[...] = jnp.full((NSLOT, Q_BLOCK, LANES), NEG_INF, jnp.float32)
        rt_ref[...] = jnp.zeros((NSLOT, Q_BLOCK, LANES), jnp.int32)

    sim = jnp.dot(qn_ref[...], e_ref[...], preferred_element_type=jnp.float32)
    lane = jax.lax.broadcasted_iota(jnp.int32, (Q_BLOCK, LANES), 1)

    rv = [rv_ref[s] for s in range(NSLOT)]
    rt = [rt_ref[s] for s in range(NSLOT)]

    def sorted_quad(qd):
        vs, ts = [], []
        for j in range(4):
            g = qd * 4 + j
            tag = k * GROUPS + g
            v = sim[:, g * LANES:(g + 1) * LANES]
            v = jnp.where(lane < k_total - tag * LANES, v, NEG_INF)
            vs.append(v)
            ts.append(jnp.full((Q_BLOCK, LANES), tag, jnp.int32))

        def comp(i, j):
            c = vs[j] > vs[i]
            hi_v, lo_v = jnp.maximum(vs[i], vs[j]), jnp.minimum(vs[i], vs[j])
            hi_t = jnp.where(c, ts[j], ts[i])
            lo_t = jnp.where(c, ts[i], ts[j])
            vs[i], vs[j] = hi_v, lo_v
            ts[i], ts[j] = hi_t, lo_t

        comp(0, 1); comp(2, 3); comp(0, 2); comp(1, 3); comp(1, 2)
        return vs, ts

    def play(av, at, bv, bt):
        c = bv > av
        return (jnp.maximum(av, bv), jnp.where(c, bt, at),
                jnp.minimum(av, bv), jnp.where(c, at, bt))

    for pr in range(GROUPS // 16):
        q = [sorted_quad(4 * pr + j) for j in range(4)]
        w0v, w0t, l0v, l0t = play(q[0][0][0], q[0][1][0], q[1][0][0], q[1][1][0])
        w1v, w1t, l1v, l1t = play(q[2][0][0], q[2][1][0], q[3][0][0], q[3][1][0])
        cv_, ct_, mv_, mt_ = play(w0v, w0t, w1v, w1t)
        _insert(rv, rt, 0, 8, cv_, ct_)
        _insert(rv, rt, 8, 4, l0v, l0t)
        _insert(rv, rt, 8, 4, l1v, l1t)
        _insert(rv, rt, 12, 4, mv_, mt_)
        b0v, b0t, n0v, n0t = play(q[0][0][1], q[0][1][1], q[1][0][1], q[1][1][1])
        b1v, b1t, n1v, n1t = play(q[2][0][1], q[2][1][1], q[3][0][1], q[3][1][1])
        _insert(rv, rt, 16, 4, b0v, b0t)
        _insert(rv, rt, 16, 4, b1v, b1t)
        _insert(rv, rt, 20, 2, n0v, n0t)
        _insert(rv, rt, 20, 2, n1v, n1t)
        for j in range(4):
            _insert(rv, rt, 22, 2, q[j][0][2], q[j][1][2])
            _insert(rv, rt, 24, 2, q[j][0][3], q[j][1][3])
    for s in range(NSLOT):
        rv_ref[s] = rv[s]
        rt_ref[s] = rt[s]

    @pl.when(k == nk - 1)
    def _done():
        cv = [rv_ref[s] for s in range(NSLOT)]
        gi = [rt_ref[s] * LANES + lane for s in range(NSLOT)]
        for t in range(TOPK_N):
            m = cv[0]
            for s in range(1, NSLOT):
                m = jnp.maximum(m, cv[s])
            m = jnp.max(m, axis=1, keepdims=True)
            eq = [cv[s] == m for s in range(NSLOT)]
            ai = jnp.where(eq[0], gi[0], INT_MAX)
            for s in range(1, NSLOT):
                ai = jnp.minimum(ai, jnp.where(eq[s], gi[s], INT_MAX))
            ai = jnp.min(ai, axis=1, keepdims=True)
            vals_ref[:, t:t + 1] = m
            idx_ref[:, t:t + 1] = ai
            cv = [jnp.where(eq[s] & (gi[s] == ai), NEG_INF, cv[s])
                  for s in range(NSLOT)]


def kernel(queries, embeddings):
    q_total, d = queries.shape
    d2, k_total = embeddings.shape
    assert d == d2
    nq = q_total // Q_BLOCK
    nk = (k_total + K_BLOCK - 1) // K_BLOCK

    grid = (nq, nk)
    vals, idx = pl.pallas_call(
        functools.partial(_fused_body, k_total),
        grid=grid,
        in_specs=[
            pl.BlockSpec((Q_BLOCK, d), lambda q, k: (q, 0)),
            pl.BlockSpec((d, K_BLOCK), lambda q, k: (0, k)),
        ],
        out_specs=[
            pl.BlockSpec((Q_BLOCK, TOPK_N), lambda q, k: (q, 0)),
            pl.BlockSpec((Q_BLOCK, TOPK_N), lambda q, k: (q, 0)),
        ],
        out_shape=[
            jax.ShapeDtypeStruct((q_total, TOPK_N), jnp.float32),
            jax.ShapeDtypeStruct((q_total, TOPK_N), jnp.int32),
        ],
        scratch_shapes=[
            pltpu.VMEM((Q_BLOCK, d), jnp.float32),
            pltpu.VMEM((NSLOT, Q_BLOCK, LANES), jnp.float32),
            pltpu.VMEM((NSLOT, Q_BLOCK, LANES), jnp.int32),
        ],
        compiler_params=pltpu.CompilerParams(
            dimension_semantics=("arbitrary", "arbitrary"),
        ),
    )(queries, embeddings)
    return vals, idx


# speedup vs baseline: 1.0547x; 1.0547x over previous
import functools

import jax
import jax.numpy as jnp
from jax.experimental import pallas as pl
from jax.experimental.pallas import tpu as pltpu

TOPK_N = 8
Q_BLOCK = 512
K_BLOCK = 2048
LANES = 128
GROUPS = K_BLOCK // LANES
NSLOT = 20
NEG_INF = float("-inf")
INT_MAX = 2**31 - 1


def _insert(rv, rt, base, depth, v, t):
    c = [v > rv[base + s] for s in range(depth)]
    for s in range(depth - 1, 0, -1):
        rv[base + s] = jnp.where(c[s], jnp.where(c[s - 1], rv[base + s - 1], v),
                                 rv[base + s])
        rt[base + s] = jnp.where(c[s], jnp.where(c[s - 1], rt[base + s - 1], t),
                                 rt[base + s])
    rv[base] = jnp.where(c[0], v, rv[base])
    rt[base] = jnp.where(c[0], t, rt[base])


def _fused_body(k_total, q_ref, e_ref, vals_ref, idx_ref, qn_ref, rv_ref, rt_ref):
    k = pl.program_id(1)
    nk = pl.num_programs(1)

    @pl.when(k == 0)
    def _init():
        q = q_ref[...]
        denom = jnp.clip(jnp.sum(jnp.abs(q), axis=1, keepdims=True), 1e-12, None)
        qn_ref[...] = q / denom
        rv_ref[...] = jnp.full((NSLOT, Q_BLOCK, LANES), NEG_INF, jnp.float32)
        rt_ref[...] = jnp.zeros((NSLOT, Q_BLOCK, LANES), jnp.int32)

    sim = jnp.dot(qn_ref[...], e_ref[...], preferred_element_type=jnp.float32)
    lane = jax.lax.broadcasted_iota(jnp.int32, (Q_BLOCK, LANES), 1)

    rv = [rv_ref[s] for s in range(NSLOT)]
    rt = [rt_ref[s] for s in range(NSLOT)]

    def sorted_quad(qd):
        vs, ts = [], []
        for j in range(4):
            g = qd * 4 + j
            tag = k * GROUPS + g
            v = sim[:, g * LANES:(g + 1) * LANES]
            v = jnp.where(lane < k_total - tag * LANES, v, NEG_INF)
            vs.append(v)
            ts.append(jnp.full((Q_BLOCK, LANES), tag, jnp.int32))

        def comp(i, j):
            c = vs[j] > vs[i]
            hi_v, lo_v = jnp.maximum(vs[i], vs[j]), jnp.minimum(vs[i], vs[j])
            hi_t = jnp.where(c, ts[j], ts[i])
            lo_t = jnp.where(c, ts[i], ts[j])
            vs[i], vs[j] = hi_v, lo_v
            ts[i], ts[j] = hi_t, lo_t

        comp(0, 1); comp(2, 3); comp(0, 2); comp(1, 3); comp(1, 2)
        return vs, ts

    for pr in range(GROUPS // 8):
        v0, t0 = sorted_quad(2 * pr)
        v1, t1 = sorted_quad(2 * pr + 1)
        c = v1[0] > v0[0]
        w_v = jnp.maximum(v0[0], v1[0])
        l_v = jnp.minimum(v0[0], v1[0])
        w_t = jnp.where(c, t1[0], t0[0])
        l_t = jnp.where(c, t0[0], t1[0])
        _insert(rv, rt, 0, 8, w_v, w_t)
        _insert(rv, rt, 8, 4, l_v, l_t)
        _insert(rv, rt, 12, 4, v0[1], t0[1])
        _insert(rv, rt, 12, 4, v1[1], t1[1])
        _insert(rv, rt, 16, 2, v0[2], t0[2])
        _insert(rv, rt, 16, 2, v1[2], t1[2])
        _insert(rv, rt, 18, 2, v0[3], t0[3])
        _insert(rv, rt, 18, 2, v1[3], t1[3])
    for s in range(NSLOT):
        rv_ref[s] = rv[s]
        rt_ref[s] = rt[s]

    @pl.when(k == nk - 1)
    def _done():
        cv = [rv_ref[s] for s in range(NSLOT)]
        gi = [rt_ref[s] * LANES + lane for s in range(NSLOT)]
        for t in range(TOPK_N):
            m = cv[0]
            for s in range(1, NSLOT):
                m = jnp.maximum(m, cv[s])
            m = jnp.max(m, axis=1, keepdims=True)
            eq = [cv[s] == m for s in range(NSLOT)]
            ai = jnp.where(eq[0], gi[0], INT_MAX)
            for s in range(1, NSLOT):
                ai = jnp.minimum(ai, jnp.where(eq[s], gi[s], INT_MAX))
            ai = jnp.min(ai, axis=1, keepdims=True)
            vals_ref[:, t:t + 1] = m
            idx_ref[:, t:t + 1] = ai
            cv = [jnp.where(eq[s] & (gi[s] == ai), NEG_INF, cv[s])
                  for s in range(NSLOT)]


def kernel(queries, embeddings):
    q_total, d = queries.shape
    d2, k_total = embeddings.shape
    assert d == d2
    nq = q_total // Q_BLOCK
    nk = (k_total + K_BLOCK - 1) // K_BLOCK

    grid = (nq, nk)
    vals, idx = pl.pallas_call(
        functools.partial(_fused_body, k_total),
        grid=grid,
        in_specs=[
            pl.BlockSpec((Q_BLOCK, d), lambda q, k: (q, 0)),
            pl.BlockSpec((d, K_BLOCK), lambda q, k: (0, k)),
        ],
        out_specs=[
            pl.BlockSpec((Q_BLOCK, TOPK_N), lambda q, k: (q, 0)),
            pl.BlockSpec((Q_BLOCK, TOPK_N), lambda q, k: (q, 0)),
        ],
        out_shape=[
            jax.ShapeDtypeStruct((q_total, TOPK_N), jnp.float32),
            jax.ShapeDtypeStruct((q_total, TOPK_N), jnp.int32),
        ],
        scratch_shapes=[
            pltpu.VMEM((Q_BLOCK, d), jnp.float32),
            pltpu.VMEM((NSLOT, Q_BLOCK, LANES), jnp.float32),
            pltpu.VMEM((NSLOT, Q_BLOCK, LANES), jnp.int32),
        ],
        compiler_params=pltpu.CompilerParams(
            dimension_semantics=("arbitrary", "arbitrary"),
        ),
    )(queries, embeddings)
    return vals, idx
